# Initial kernel scaffold; baseline (speedup 1.0000x reference)
import jax
import jax.numpy as jnp
from jax.experimental import pallas as pl


def kernel(atomic_numbers, coeff_ind_to_node_ind, emb_weight):
    raise NotImplementedError("write your pallas kernel here")


# speedup vs baseline: 20.8814x; 20.8814x over previous
import functools

import jax
import jax.numpy as jnp
from jax import lax
from jax.experimental import pallas as pl
from jax.experimental.pallas import tpu as pltpu
from jax.experimental.pallas import tpu_sc as plsc

NC, NS, L = 2, 16, 16
NW = NC * NS
CNT_H, CNT_X = 5, 14
MAXB = 14
MAXS = 3
SDIMS = (2, 3, 3, 3)
CHN = 64
TROWS = 4 * MAXB
BLK = 128


def _prep_body(emb_ref, tab_ref, emb_v, tab_v):
    wid = lax.axis_index("s") * NC + lax.axis_index("c")

    @pl.when(wid == 0)
    def _():
        pltpu.sync_copy(emb_ref, emb_v)
        zeros = jnp.zeros((L,), jnp.float32)
        for t in range(4):
            for p in range(MAXB):
                for g in range(CHN // L):
                    if p < SDIMS[t]:
                        val = emb_v[pl.ds(t * MAXS * CHN + p * CHN + g * L, L)]
                    else:
                        val = zeros
                    tab_v[pl.ds((t * MAXB + p) * CHN + g * L, L)] = val
        pltpu.sync_copy(tab_v, tab_ref)


def _cnt_of(vec):
    return jnp.where(vec == 1, CNT_H, CNT_X).astype(jnp.int32)


def _main_body(tch, nch, nv, an_ref, cf_ref, tab_ref, out_ref,
               an_v, b_v, cf_v, key_v, rows_v, sem):
    wid = lax.axis_index("s") * NC + lax.axis_index("c")
    s0 = wid * tch

    pltpu.sync_copy(an_ref, an_v)

    CO = 8

    def blk_total(m):
        acc = jnp.zeros((L,), jnp.int32)
        for u in range(CO):
            acc = acc + _cnt_of(an_v[pl.ds((m * CO + u) * L, L)])
        return jnp.sum(acc)

    def c_cond(st):
        _, off, ct = st
        return off + ct <= s0

    def c_body(st):
        m, off, ct = st
        return (m + 1, off + ct, blk_total(m + 1))

    m_c, off_c, _ = lax.while_loop(c_cond, c_body, (0, 0, blk_total(0)))

    def vec_total(k):
        return jnp.sum(_cnt_of(an_v[pl.ds(k * L, L)]))

    def f_cond(st):
        _, off, ct = st
        return off + ct <= s0

    def f_body(st):
        k, off, ct = st
        return (k + 1, off + ct, vec_total(k + 1))

    k0 = m_c * CO
    k_f, off_f, _ = lax.while_loop(f_cond, f_body, (k0, off_c, vec_total(k0)))
    nb = k_f * L

    def p2_body(j, off):
        vec = an_v[pl.ds((k_f + j) * L, L)]
        cnt = _cnt_of(vec)
        aidx = jnp.where(vec == 1, 0,
                         jnp.where(vec == 6, 1,
                                   jnp.where(vec == 7, 2, 3))).astype(jnp.int32)
        cs = plsc.cumsum(cnt)
        b_v[pl.ds(j * L, L)] = aidx * MAXB - (off + cs - cnt)
        return off + jnp.sum(cnt)

    lax.fori_loop(0, nv, p2_body, off_f)

    iota16 = lax.iota(jnp.int32, L)
    loc_max = nv * L - 1

    def pb_body(j, carry):
        base = s0 + j * BLK
        pltpu.sync_copy(cf_ref.at[pl.ds(base, BLK)], cf_v)
        for g in range(BLK // L):
            c = cf_v[pl.ds(g * L, L)]
            loc = jnp.clip(c - nb, 0, loc_max)
            bg = plsc.load_gather(b_v, [loc])
            key = bg + (base + g * L) + iota16
            key_v[pl.ds(g * L, L)] = jnp.clip(key, 0, TROWS - 1)
        pltpu.async_copy(tab_ref.at[key_v], rows_v, sem).wait()
        pltpu.sync_copy(rows_v, out_ref.at[pl.ds(base, BLK)])
        return carry

    lax.fori_loop(0, nch, pb_body, 0)


def kernel(atomic_numbers, coeff_ind_to_node_ind, emb_weight):
    n = atomic_numbers.shape[0]
    t = coeff_ind_to_node_ind.shape[0]
    tch = -(-t // (NW * BLK)) * BLK
    tpad = NW * tch
    nch = tch // BLK
    nv = (tch // CNT_H + 64) // L + 2
    anp = -(-n // L) * L + nv * L

    an_p = jnp.pad(atomic_numbers.astype(jnp.int32), (0, anp - n),
                   constant_values=8)
    cf_p = jnp.pad(coeff_ind_to_node_ind.astype(jnp.int32), (0, tpad - t))

    mesh = plsc.VectorSubcoreMesh(core_axis_name="c", subcore_axis_name="s")
    cparams = pltpu.CompilerParams(needs_layout_passes=False,
                                   use_tc_tiling_on_sc=False)

    tab_flat = pl.kernel(
        _prep_body,
        out_type=jax.ShapeDtypeStruct((TROWS * CHN,), jnp.float32),
        mesh=mesh,
        compiler_params=cparams,
        scratch_types=[
            pltpu.VMEM((4 * MAXS * CHN,), jnp.float32),
            pltpu.VMEM((TROWS * CHN,), jnp.float32),
        ],
    )(emb_weight.reshape(-1))
    table = tab_flat.reshape(TROWS, CHN)

    out_p = pl.kernel(
        functools.partial(_main_body, tch, nch, nv),
        out_type=jax.ShapeDtypeStruct((tpad, CHN), jnp.float32),
        mesh=mesh,
        compiler_params=cparams,
        scratch_types=[
            pltpu.VMEM((anp,), jnp.int32),
            pltpu.VMEM((nv * L,), jnp.int32),
            pltpu.VMEM((BLK,), jnp.int32),
            pltpu.VMEM((BLK,), jnp.int32),
            pltpu.VMEM((BLK, CHN), jnp.float32),
            pltpu.SemaphoreType.DMA,
        ],
    )(an_p, cf_p, table)
    return out_p[:t]


# speedup vs baseline: 21.2307x; 1.0167x over previous
import functools

import jax
import jax.numpy as jnp
from jax import lax
from jax.experimental import pallas as pl
from jax.experimental.pallas import tpu as pltpu
from jax.experimental.pallas import tpu_sc as plsc

NC, NS, L = 2, 16, 16
NW = NC * NS
CNT_H, CNT_X = 5, 14
MAXB = 14
MAXS = 3
SDIMS = (2, 3, 3, 3)
CHN = 64
TROWS = 4 * MAXB
BLK = 128
NBUF = 4


def _prep_body(emb_ref, tab_ref, emb_v, tab_v):
    wid = lax.axis_index("s") * NC + lax.axis_index("c")

    @pl.when(wid == 0)
    def _():
        pltpu.sync_copy(emb_ref, emb_v)
        zeros = jnp.zeros((L,), jnp.float32)
        for t in range(4):
            for p in range(MAXB):
                for g in range(CHN // L):
                    if p < SDIMS[t]:
                        val = emb_v[pl.ds(t * MAXS * CHN + p * CHN + g * L, L)]
                    else:
                        val = zeros
                    tab_v[pl.ds((t * MAXB + p) * CHN + g * L, L)] = val
        pltpu.sync_copy(tab_v, tab_ref)


def _cnt_of(vec):
    return jnp.where(vec == 1, CNT_H, CNT_X).astype(jnp.int32)


def _main_body(tch, nch, nv, an_ref, cf_ref, tab_ref, out_ref,
               an_v, b_v, cf_v, key_v, rows_v, csem, gsem, wsem):
    wid = lax.axis_index("s") * NC + lax.axis_index("c")
    s0 = wid * tch

    pltpu.sync_copy(an_ref, an_v)

    CO = 8

    def blk_total(m):
        acc = jnp.zeros((L,), jnp.int32)
        for u in range(CO):
            acc = acc + _cnt_of(an_v[pl.ds((m * CO + u) * L, L)])
        return jnp.sum(acc)

    def c_cond(st):
        _, off, ct = st
        return off + ct <= s0

    def c_body(st):
        m, off, ct = st
        return (m + 1, off + ct, blk_total(m + 1))

    m_c, off_c, _ = lax.while_loop(c_cond, c_body, (0, 0, blk_total(0)))

    def vec_total(k):
        return jnp.sum(_cnt_of(an_v[pl.ds(k * L, L)]))

    def f_cond(st):
        _, off, ct = st
        return off + ct <= s0

    def f_body(st):
        k, off, ct = st
        return (k + 1, off + ct, vec_total(k + 1))

    k0 = m_c * CO
    k_f, off_f, _ = lax.while_loop(f_cond, f_body, (k0, off_c, vec_total(k0)))
    nb = k_f * L

    def p2_body(j, off):
        vec = an_v[pl.ds((k_f + j) * L, L)]
        cnt = _cnt_of(vec)
        aidx = jnp.where(vec == 1, 0,
                         jnp.where(vec == 6, 1,
                                   jnp.where(vec == 7, 2, 3))).astype(jnp.int32)
        cs = plsc.cumsum(cnt)
        b_v[pl.ds(j * L, L)] = aidx * MAXB - (off + cs - cnt)
        return off + jnp.sum(cnt)

    lax.fori_loop(0, nv, p2_body, off_f)

    iota16 = lax.iota(jnp.int32, L)
    loc_max = nv * L - 1
    ng = nch // NBUF

    def cf_copy(j, b):
        return pltpu.make_async_copy(
            cf_ref.at[pl.ds(s0 + j * BLK, BLK)], cf_v.at[b], csem.at[b])

    def g_copy(b):
        return pltpu.make_async_copy(
            tab_ref.at[key_v.at[b]], rows_v.at[b], gsem.at[b])

    def w_copy(j, b):
        return pltpu.make_async_copy(
            rows_v.at[b], out_ref.at[pl.ds(s0 + j * BLK, BLK)], wsem.at[b])

    for b in range(NBUF):
        cf_copy(b, b).start()

    def grp_body(g, carry):
        for b in range(NBUF):
            j = g * NBUF + b
            cf_copy(j, b).wait()
            for v in range(BLK // L):
                c = cf_v[b, pl.ds(v * L, L)]
                loc = jnp.clip(c - nb, 0, loc_max)
                bg = plsc.load_gather(b_v, [loc])
                key = bg + (j * BLK + s0 + v * L) + iota16
                key_v[b, pl.ds(v * L, L)] = jnp.clip(key, 0, TROWS - 1)

            @pl.when(g > 0)
            def _():
                w_copy(0, b).wait()

            g_copy(b).start()
            cf_copy(j + NBUF, b).start()

            if b == 0:
                @pl.when(g > 0)
                def _():
                    g_copy(NBUF - 1).wait()
                    w_copy(g * NBUF - 1, NBUF - 1).start()
            else:
                g_copy(b - 1).wait()
                w_copy(j - 1, b - 1).start()
        return carry

    lax.fori_loop(0, ng, grp_body, 0)

    g_copy(NBUF - 1).wait()
    w_copy(nch - 1, NBUF - 1).start()
    for b in range(NBUF):
        w_copy(0, b).wait()
        cf_copy(0, b).wait()


def kernel(atomic_numbers, coeff_ind_to_node_ind, emb_weight):
    n = atomic_numbers.shape[0]
    t = coeff_ind_to_node_ind.shape[0]
    grain = NW * BLK * NBUF
    tch = -(-t // grain) * (BLK * NBUF)
    tpad = NW * tch
    nch = tch // BLK
    nv = (tch // CNT_H + 64) // L + 2
    anp = -(-n // L) * L + nv * L

    an_p = jnp.pad(atomic_numbers.astype(jnp.int32), (0, anp - n),
                   constant_values=8)
    cf_p = jnp.pad(coeff_ind_to_node_ind.astype(jnp.int32),
                   (0, tpad + NBUF * BLK - t))

    mesh = plsc.VectorSubcoreMesh(core_axis_name="c", subcore_axis_name="s")
    cparams = pltpu.CompilerParams(needs_layout_passes=False,
                                   use_tc_tiling_on_sc=False)

    tab_flat = pl.kernel(
        _prep_body,
        out_type=jax.ShapeDtypeStruct((TROWS * CHN,), jnp.float32),
        mesh=mesh,
        compiler_params=cparams,
        scratch_types=[
            pltpu.VMEM((4 * MAXS * CHN,), jnp.float32),
            pltpu.VMEM((TROWS * CHN,), jnp.float32),
        ],
    )(emb_weight.reshape(-1))
    table = tab_flat.reshape(TROWS, CHN)

    out_p = pl.kernel(
        functools.partial(_main_body, tch, nch, nv),
        out_type=jax.ShapeDtypeStruct((tpad, CHN), jnp.float32),
        mesh=mesh,
        compiler_params=cparams,
        scratch_types=[
            pltpu.VMEM((anp,), jnp.int32),
            pltpu.VMEM((nv * L,), jnp.int32),
            pltpu.VMEM((NBUF, BLK), jnp.int32),
            pltpu.VMEM((NBUF, BLK), jnp.int32),
            pltpu.VMEM((NBUF, BLK, CHN), jnp.float32),
            pltpu.SemaphoreType.DMA((NBUF,)),
            pltpu.SemaphoreType.DMA((NBUF,)),
            pltpu.SemaphoreType.DMA((NBUF,)),
        ],
    )(an_p, cf_p, table)
    return out_p[:t]


# speedup vs baseline: 64.7950x; 3.0519x over previous
import functools

import jax
import jax.numpy as jnp
from jax import lax
from jax.experimental import pallas as pl
from jax.experimental.pallas import tpu as pltpu
from jax.experimental.pallas import tpu_sc as plsc

NC, NS, L = 2, 16, 16
NW = NC * NS
CNT_H, CNT_X = 5, 14
MAXB = 14
MAXS = 3
SDIMS = (2, 3, 3, 3)
CHN = 64
TROWS = 4 * MAXB
RING = 48


def _cnt_of(vec):
    return jnp.where(vec == 1, CNT_H, CNT_X).astype(jnp.int32)


def _main_body(tch, nv, t_real, an_ref, emb_ref, out_ref,
               an_v, b_v, emb_v, tab_v, semh, semx):
    wid = lax.axis_index("s") * NC + lax.axis_index("c")
    s0 = wid * tch
    s1 = s0 + tch

    pltpu.sync_copy(emb_ref, emb_v)
    zeros = jnp.zeros((L,), jnp.float32)
    for t in range(4):
        for p in range(MAXB):
            for g in range(CHN // L):
                if p < SDIMS[t]:
                    val = emb_v[pl.ds(t * MAXS * CHN + p * CHN + g * L, L)]
                else:
                    val = zeros
                tab_v[t * MAXB + p, pl.ds(g * L, L)] = val

    pltpu.sync_copy(an_ref, an_v)

    CO = 8

    def blk_total(m):
        acc = jnp.zeros((L,), jnp.int32)
        for u in range(CO):
            acc = acc + _cnt_of(an_v[pl.ds((m * CO + u) * L, L)])
        return jnp.sum(acc)

    def c_cond(st):
        _, off, ct = st
        return off + ct <= s0

    def c_body(st):
        m, off, ct = st
        return (m + 1, off + ct, blk_total(m + 1))

    m_c, off_c, _ = lax.while_loop(c_cond, c_body, (0, 0, blk_total(0)))

    def vec_total(k):
        return jnp.sum(_cnt_of(an_v[pl.ds(k * L, L)]))

    def f_cond(st):
        _, off, ct = st
        return off + ct <= s0

    def f_body(st):
        k, off, ct = st
        return (k + 1, off + ct, vec_total(k + 1))

    k0 = m_c * CO
    k_f, off_f, _ = lax.while_loop(f_cond, f_body, (k0, off_c, vec_total(k0)))
    nb = k_f * L

    def p2_body(j, off):
        vec = an_v[pl.ds((k_f + j) * L, L)]
        cnt = _cnt_of(vec)
        aidx = jnp.where(vec == 1, 0,
                         jnp.where(vec == 6, 1,
                                   jnp.where(vec == 7, 2, 3))).astype(jnp.int32)
        cs = plsc.cumsum(cnt)
        b_v[pl.ds(j * L, L)] = aidx * (1 << 24) + (off + cs - cnt)
        return off + jnp.sum(cnt)

    lax.fori_loop(0, nv, p2_body, off_f)

    def h_desc(src_row, dst_row):
        return pltpu.make_async_copy(
            tab_v.at[pl.ds(src_row, CNT_H)],
            out_ref.at[pl.ds(dst_row, CNT_H)], semh)

    def x_desc(src_row, dst_row):
        return pltpu.make_async_copy(
            tab_v.at[pl.ds(src_row, CNT_X)],
            out_ref.at[pl.ds(dst_row, CNT_X)], semx)

    s1e = jnp.minimum(s1, t_real)

    def v_cond(st):
        _, _, _, vec = st
        return (vec[0] & 0x00FFFFFF) < s1

    def v_body(st):
        jv, nh, nx, vec = st
        for lane in range(L):
            code = vec[lane]
            off = code & 0x00FFFFFF
            atom = code >> 24
            is_h = atom == 0
            own = (off >= s0) & (off < s1e)

            @pl.when(own & is_h)
            def _():
                h_desc(0, off).start()

            @pl.when(own & jnp.logical_not(is_h))
            def _():
                x_desc(atom * MAXB, off).start()

            nh = nh + jnp.where(own & is_h, 1, 0)
            nx = nx + jnp.where(own & jnp.logical_not(is_h), 1, 0)

        def wh_body(c):
            h_desc(0, 0).wait()
            return c - 1

        def wx_body(c):
            x_desc(0, 0).wait()
            return c - 1

        def over(c):
            return c > RING

        nh = lax.while_loop(over, wh_body, nh)
        nx = lax.while_loop(over, wx_body, nx)
        return (jv + 1, nh, nx, b_v[pl.ds((jv + 1) * L, L)])

    _, nh, nx, _ = lax.while_loop(
        v_cond, v_body,
        (jnp.int32(0), jnp.int32(0), jnp.int32(0), b_v[pl.ds(0, L)]))

    def dh_cond(c):
        return c > 0

    def dh_body(c):
        h_desc(0, 0).wait()
        return c - 1

    lax.while_loop(dh_cond, dh_body, nh)

    def dx_body(c):
        x_desc(0, 0).wait()
        return c - 1

    lax.while_loop(dh_cond, dx_body, nx)


def kernel(atomic_numbers, coeff_ind_to_node_ind, emb_weight):
    n = atomic_numbers.shape[0]
    t = coeff_ind_to_node_ind.shape[0]
    tch = -(-t // NW)
    nv = (tch // CNT_H + 64) // L + 2
    anp = -(-n // L) * L + nv * L

    an_p = jnp.pad(atomic_numbers.astype(jnp.int32), (0, anp - n),
                   constant_values=8)

    mesh = plsc.VectorSubcoreMesh(core_axis_name="c", subcore_axis_name="s")
    cparams = pltpu.CompilerParams(needs_layout_passes=False,
                                   use_tc_tiling_on_sc=False)

    out_p = pl.kernel(
        functools.partial(_main_body, tch, nv, t),
        out_type=jax.ShapeDtypeStruct((t, CHN), jnp.float32),
        mesh=mesh,
        compiler_params=cparams,
        scratch_types=[
            pltpu.VMEM((anp,), jnp.int32),
            pltpu.VMEM(((nv + 1) * L,), jnp.int32),
            pltpu.VMEM((4 * MAXS * CHN,), jnp.float32),
            pltpu.VMEM((TROWS, CHN), jnp.float32),
            pltpu.SemaphoreType.DMA,
            pltpu.SemaphoreType.DMA,
        ],
    )(an_p, emb_weight.reshape(-1))
    return out_p
